# Initial kernel scaffold; baseline (speedup 1.0000x reference)
import jax
import jax.numpy as jnp
from jax.experimental import pallas as pl


def kernel(x, fc1_w, fc1_b, fc2_w, fc2_b, expert_bias, router_w):
    raise NotImplementedError("write your pallas kernel here")


# speedup vs baseline: 1.8925x; 1.8925x over previous
import jax
import jax.numpy as jnp
from jax.experimental import pallas as pl

N_TOKENS = 8192
HIDDEN = 2048
PRED_H = 256
N_EXPERTS = 64
N_CAND = 16
BT = 512


def _router_body(x_ref, w1t_ref, b1_ref, w2t_ref, b2_ref, rwt_ref,
                 w_out_ref, id_out_ref):
    x = x_ref[...]
    h = jnp.maximum(
        jnp.dot(x, w1t_ref[...], preferred_element_type=jnp.float32)
        + b1_ref[...], 0.0)
    p = (jnp.dot(h, w2t_ref[...], preferred_element_type=jnp.float32)
         + b2_ref[...])
    f = jnp.dot(x, rwt_ref[...], preferred_element_type=jnp.float32)

    cur = p
    for _ in range(N_CAND - 1):
        m = jnp.max(cur, axis=1, keepdims=True)
        cur = jnp.where(cur >= m, -jnp.inf, cur)
    thresh = jnp.max(cur, axis=1, keepdims=True)

    neg = jnp.float32(-1e9)
    g = jnp.where(p >= thresh, f, neg)
    iota = jax.lax.broadcasted_iota(jnp.int32, g.shape, 1)
    v1 = jnp.max(g, axis=1, keepdims=True)
    i1 = jnp.min(jnp.where(g >= v1, iota, N_EXPERTS), axis=1, keepdims=True)
    g2 = jnp.where(iota == i1, neg, g)
    v2 = jnp.max(g2, axis=1, keepdims=True)
    i2 = jnp.min(jnp.where(g2 >= v2, iota, N_EXPERTS), axis=1, keepdims=True)

    e = jnp.exp(v2 - v1)
    inv = 1.0 / (1.0 + e)
    w_out_ref[...] = jnp.concatenate([inv, e * inv], axis=1)
    id_out_ref[...] = jnp.concatenate([i1, i2], axis=1)


def kernel(x, fc1_w, fc1_b, fc2_w, fc2_b, expert_bias, router_w):
    w1t = fc1_w.T
    w2t = fc2_w.T
    b1 = fc1_b.reshape(1, PRED_H)
    b2 = (fc2_b + expert_bias).reshape(1, N_EXPERTS)
    rwt = router_w.T

    grid = (N_TOKENS // BT,)
    out_w, out_id = pl.pallas_call(
        _router_body,
        grid=grid,
        in_specs=[
            pl.BlockSpec((BT, HIDDEN), lambda i: (i, 0)),
            pl.BlockSpec((HIDDEN, PRED_H), lambda i: (0, 0)),
            pl.BlockSpec((1, PRED_H), lambda i: (0, 0)),
            pl.BlockSpec((PRED_H, N_EXPERTS), lambda i: (0, 0)),
            pl.BlockSpec((1, N_EXPERTS), lambda i: (0, 0)),
            pl.BlockSpec((HIDDEN, N_EXPERTS), lambda i: (0, 0)),
        ],
        out_specs=[
            pl.BlockSpec((BT, 2), lambda i: (i, 0)),
            pl.BlockSpec((BT, 2), lambda i: (i, 0)),
        ],
        out_shape=[
            jax.ShapeDtypeStruct((N_TOKENS, 2), jnp.float32),
            jax.ShapeDtypeStruct((N_TOKENS, 2), jnp.int32),
        ],
    )(x, w1t, b1, w2t, b2, rwt)
    return out_w, out_id


# speedup vs baseline: 2.0953x; 1.1072x over previous
import jax
import jax.numpy as jnp
from jax.experimental import pallas as pl
from jax.experimental.pallas import tpu as pltpu

N_TOKENS = 8192
HIDDEN = 2048
PRED_H = 256
N_EXPERTS = 64
N_CAND = 16
BT = 512
NB = N_TOKENS // BT


def _router_body(x_ref, w1t_ref, b1_ref, w2t_ref, b2_ref, rwt_ref,
                 w_out_ref, id_out_ref, p_scr, f_scr):
    i = pl.program_id(0)

    tslot = jax.lax.rem(i + 1, 2)
    p = p_scr[tslot]
    f = f_scr[tslot]

    cur = p
    for _ in range(N_CAND - 1):
        m = jnp.max(cur, axis=1, keepdims=True)
        cur = jnp.where(cur >= m, -jnp.inf, cur)
    thresh = jnp.max(cur, axis=1, keepdims=True)

    neg = jnp.float32(-1e9)
    g = jnp.where(p >= thresh, f, neg)
    iota = jax.lax.broadcasted_iota(jnp.int32, g.shape, 1)
    v1 = jnp.max(g, axis=1, keepdims=True)
    i1 = jnp.min(jnp.where(g >= v1, iota, N_EXPERTS), axis=1,
                 keepdims=True)
    g2 = jnp.where(iota == i1, neg, g)
    v2 = jnp.max(g2, axis=1, keepdims=True)
    i2 = jnp.min(jnp.where(g2 >= v2, iota, N_EXPERTS), axis=1,
                 keepdims=True)

    e = jnp.exp(v2 - v1)
    inv = 1.0 / (1.0 + e)
    w_out_ref[...] = jnp.concatenate([inv, e * inv], axis=1)
    id_out_ref[...] = jnp.concatenate([i1, i2], axis=1)

    x = x_ref[...]
    h = jnp.maximum(
        jnp.dot(x, w1t_ref[...], preferred_element_type=jnp.float32)
        + b1_ref[...], 0.0)
    slot = jax.lax.rem(i, 2)
    p_scr[slot] = (jnp.dot(h, w2t_ref[...],
                           preferred_element_type=jnp.float32)
                   + b2_ref[...])
    f_scr[slot] = jnp.dot(x, rwt_ref[...],
                          preferred_element_type=jnp.float32)


def kernel(x, fc1_w, fc1_b, fc2_w, fc2_b, expert_bias, router_w):
    w1t = fc1_w.T
    w2t = fc2_w.T
    b1 = fc1_b.reshape(1, PRED_H)
    b2 = (fc2_b + expert_bias).reshape(1, N_EXPERTS)
    rwt = router_w.T

    out_w, out_id = pl.pallas_call(
        _router_body,
        grid=(NB + 1,),
        in_specs=[
            pl.BlockSpec((BT, HIDDEN), lambda i: (jnp.minimum(i, NB - 1), 0)),
            pl.BlockSpec((HIDDEN, PRED_H), lambda i: (0, 0)),
            pl.BlockSpec((1, PRED_H), lambda i: (0, 0)),
            pl.BlockSpec((PRED_H, N_EXPERTS), lambda i: (0, 0)),
            pl.BlockSpec((1, N_EXPERTS), lambda i: (0, 0)),
            pl.BlockSpec((HIDDEN, N_EXPERTS), lambda i: (0, 0)),
        ],
        out_specs=[
            pl.BlockSpec((BT, 2), lambda i: (jnp.maximum(i - 1, 0), 0)),
            pl.BlockSpec((BT, 2), lambda i: (jnp.maximum(i - 1, 0), 0)),
        ],
        out_shape=[
            jax.ShapeDtypeStruct((N_TOKENS, 2), jnp.float32),
            jax.ShapeDtypeStruct((N_TOKENS, 2), jnp.int32),
        ],
        scratch_shapes=[
            pltpu.VMEM((2, BT, N_EXPERTS), jnp.float32),
            pltpu.VMEM((2, BT, N_EXPERTS), jnp.float32),
        ],
    )(x, w1t, b1, w2t, b2, rwt)
    return out_w, out_id


# speedup vs baseline: 2.3598x; 1.1262x over previous
import jax
import jax.numpy as jnp
from jax.experimental import pallas as pl
from jax.experimental.pallas import tpu as pltpu

N_TOKENS = 8192
HIDDEN = 2048
PRED_H = 256
N_EXPERTS = 64
N_CAND = 16
BT = 512
CHUNK = 128
NB = N_TOKENS // BT


def _tdot(a, b):
    return jax.lax.dot_general(a, b, (((1,), (1,)), ((), ())),
                               preferred_element_type=jnp.float32)


def _router_body(x_ref, w1_ref, b1_ref, w2_ref, b2_ref, eb_ref, rw_ref,
                 w_out_ref, id_out_ref, p_scr, f_scr):
    i = pl.program_id(0)

    tslot = jax.lax.rem(i + 1, 2)
    neg = jnp.float32(-1e9)
    cur = p_scr[tslot]
    for _ in range(N_CAND - 1):
        m = jnp.max(cur, axis=1, keepdims=True)
        cur = jnp.where(cur >= m, -jnp.inf, cur)
    thresh = jnp.max(cur, axis=1, keepdims=True)

    f = f_scr[tslot]
    cand = jnp.logical_or(cur == -jnp.inf, cur >= thresh)
    g = jnp.where(cand, f, neg)
    iota = jax.lax.broadcasted_iota(jnp.int32, g.shape, 1).astype(jnp.float32)
    big = jnp.float32(N_EXPERTS)
    v1 = jnp.max(g, axis=1, keepdims=True)
    i1 = jnp.min(jnp.where(g >= v1, iota, big), axis=1, keepdims=True)
    g2 = jnp.where(iota == i1, neg, g)
    v2 = jnp.max(g2, axis=1, keepdims=True)
    i2 = jnp.min(jnp.where(g2 >= v2, iota, big), axis=1, keepdims=True)

    e = jnp.exp(v2 - v1)
    inv = 1.0 / (1.0 + e)
    zf = jnp.zeros_like(g)
    w_out_ref[...] = jnp.where(iota == 0, inv,
                               jnp.where(iota == 1, e * inv, zf))
    id_out_ref[...] = jnp.where(iota == 0, i1,
                                jnp.where(iota == 1, i2, zf))

    x = x_ref[...]
    h = jnp.maximum(_tdot(x, w1_ref[...]) + b1_ref[...], 0.0)
    slot = jax.lax.rem(i, 2)
    p_scr[slot] = _tdot(h, w2_ref[...]) + (b2_ref[...] + eb_ref[...])
    f_scr[slot] = _tdot(x, rw_ref[...])


def kernel(x, fc1_w, fc1_b, fc2_w, fc2_b, expert_bias, router_w):
    b1 = fc1_b.reshape(1, PRED_H)
    b2 = fc2_b.reshape(1, N_EXPERTS)
    eb = expert_bias.reshape(1, N_EXPERTS)

    out_w, out_id = pl.pallas_call(
        _router_body,
        grid=(NB + 1,),
        in_specs=[
            pl.BlockSpec((BT, HIDDEN), lambda i: (jnp.minimum(i, NB - 1), 0)),
            pl.BlockSpec((PRED_H, HIDDEN), lambda i: (0, 0)),
            pl.BlockSpec((1, PRED_H), lambda i: (0, 0)),
            pl.BlockSpec((N_EXPERTS, PRED_H), lambda i: (0, 0)),
            pl.BlockSpec((1, N_EXPERTS), lambda i: (0, 0)),
            pl.BlockSpec((1, N_EXPERTS), lambda i: (0, 0)),
            pl.BlockSpec((N_EXPERTS, HIDDEN), lambda i: (0, 0)),
        ],
        out_specs=[
            pl.BlockSpec((BT, N_EXPERTS), lambda i: (jnp.maximum(i - 1, 0), 0)),
            pl.BlockSpec((BT, N_EXPERTS), lambda i: (jnp.maximum(i - 1, 0), 0)),
        ],
        out_shape=[
            jax.ShapeDtypeStruct((N_TOKENS, N_EXPERTS), jnp.float32),
            jax.ShapeDtypeStruct((N_TOKENS, N_EXPERTS), jnp.float32),
        ],
        scratch_shapes=[
            pltpu.VMEM((2, BT, N_EXPERTS), jnp.float32),
            pltpu.VMEM((2, BT, N_EXPERTS), jnp.float32),
        ],
    )(x, fc1_w, b1, fc2_w, b2, eb, router_w)
    return out_w[:, :2], out_id[:, :2].astype(jnp.int32)
